# baseline (speedup 1.0000x reference)
import functools

import jax
import jax.numpy as jnp
from jax import lax
from jax.experimental import pallas as pl
from jax.experimental.pallas import tpu as pltpu
from jax.experimental.pallas import tpu_sc as plsc

NC = 2
NS = 16
LANES = 16
ROW_W = 8
CHUNK_ROWS = 16


@functools.lru_cache(maxsize=None)
def _make_kernel(n_rows: int):
    n_workers = NC * NS
    assert n_rows % CHUNK_ROWS == 0
    n_chunks = n_rows // CHUNK_ROWS
    ent = CHUNK_ROWS * 128
    steps = ent // (2 * LANES)
    max_mine = n_chunks // n_workers + (1 if n_chunks % n_workers else 0)
    n_it = (max_mine + 1) // 2

    mesh = plsc.VectorSubcoreMesh(core_axis_name="c", subcore_axis_name="s")

    @functools.partial(
        pl.kernel,
        out_type=jax.ShapeDtypeStruct((2 * 8,), jnp.float32),
        mesh=mesh,
        compiler_params=pltpu.CompilerParams(
            needs_layout_passes=False, use_tc_tiling_on_sc=False
        ),
        scratch_types=[
            pltpu.VMEM((2 * ent,), jnp.int32),
            pltpu.VMEM((2 * ent, ROW_W), jnp.float32),
            pltpu.VMEM((32,), jnp.float32),
            pltpu.VMEM((LANES,), jnp.float32),
            pltpu.VMEM_SHARED((NS * LANES,), jnp.float32),
            pltpu.VMEM((NS * LANES,), jnp.float32),
            pltpu.SemaphoreType.DMA,
            pltpu.SemaphoreType.DMA,
            pltpu.SemaphoreType.DMA,
            pltpu.SemaphoreType.DMA,
        ],
    )
    def lj(table, prows, params, out, idxv, rowsv, pv, accv, shared, redv,
           isem0, isem1, gsem0, gsem1):
        cid = lax.axis_index("c")
        sid = lax.axis_index("s")
        wid = sid * NC + cid
        isem = (isem0, isem1)
        gsem = (gsem0, gsem1)

        pltpu.sync_copy(params, pv)
        pva = pv[pl.ds(0, LANES)]
        pvb = pv[pl.ds(LANES, LANES)]
        bi = [pva[i] for i in range(9)]
        bx = [pva[9 + i] for i in range(7)] + [pvb[0], pvb[1]]
        c2 = pvb[2]
        thr = pvb[3]

        iota = lax.iota(jnp.int32, LANES)

        n_mine = n_chunks // n_workers + jnp.where(
            wid < n_chunks % n_workers, 1, 0
        )

        def fire_idx(t, slot):
            @pl.when(t < n_mine)
            def _():
                g = wid + t * n_workers
                pltpu.async_copy(
                    prows.at[pl.ds(g * ent, ent)],
                    idxv.at[pl.ds(slot * ent, ent)],
                    isem[slot],
                )

        def fire_gath(t, slot):
            @pl.when(t < n_mine)
            def _():
                pltpu.make_async_copy(
                    prows.at[pl.ds(0, ent)],
                    idxv.at[pl.ds(slot * ent, ent)],
                    isem[slot],
                ).wait()
                for k in range(CHUNK_ROWS):
                    pltpu.async_copy(
                        table.at[idxv.at[pl.ds(slot * ent + k * 128, 128)]],
                        rowsv.at[pl.ds(slot * ent + k * 128, 128)],
                        gsem[slot],
                    )

        def wait_gath(t, slot):
            @pl.when(t < n_mine)
            def _():
                pltpu.make_async_copy(
                    table.at[pl.ds(0, ent)],
                    rowsv.at[pl.ds(slot * ent, ent)],
                    gsem[slot],
                ).wait()

        def compute(slot):
            base = slot * ent

            def block(m, a):
                base_b = base + m * 256
                for u in range(8):
                    a = step(base_b + u * LANES, a)
                return a

            def step(b0, a):
                ri = iota + b0
                rj = ri + 128
                c0 = jnp.zeros((LANES,), jnp.int32)
                xi = plsc.load_gather(rowsv, [ri, c0])
                yi = plsc.load_gather(rowsv, [ri, c0 + 1])
                zi = plsc.load_gather(rowsv, [ri, c0 + 2])
                si = plsc.load_gather(rowsv, [ri, c0 + 3])
                ei = plsc.load_gather(rowsv, [ri, c0 + 4])
                xj = plsc.load_gather(rowsv, [rj, c0])
                yj = plsc.load_gather(rowsv, [rj, c0 + 1])
                zj = plsc.load_gather(rowsv, [rj, c0 + 2])
                sj = plsc.load_gather(rowsv, [rj, c0 + 3])
                ej = plsc.load_gather(rowsv, [rj, c0 + 4])

                dx = xi - xj
                dy = yi - yj
                dz = zi - zj

                def slow_r2():
                    d0 = dx * bi[0] + dy * bi[3] + dz * bi[6]
                    d1 = dx * bi[1] + dy * bi[4] + dz * bi[7]
                    d2 = dx * bi[2] + dy * bi[5] + dz * bi[8]

                    def wrap(v):
                        y = v + 0.5
                        tf = y.astype(jnp.int32).astype(jnp.float32)
                        fl = tf - jnp.where(y < tf, 1.0, 0.0)
                        return v - fl

                    w0 = wrap(d0)
                    w1 = wrap(d1)
                    w2 = wrap(d2)
                    rx = w0 * bx[0] + w1 * bx[3] + w2 * bx[6]
                    ry = w0 * bx[1] + w1 * bx[4] + w2 * bx[7]
                    rz = w0 * bx[2] + w1 * bx[5] + w2 * bx[8]
                    return rx * rx + ry * ry + rz * rz

                def fast_r2():
                    return dx * dx + dy * dy + dz * dz

                mx = jnp.maximum(
                    jnp.maximum(jnp.abs(dx), jnp.abs(dy)), jnp.abs(dz)
                )
                needs = jnp.any(mx >= thr)
                r2 = lax.cond(needs, slow_r2, fast_r2)

                sij = si + sj
                eij = ei * ej
                tq = (sij * sij) / r2
                t3 = tq * tq * tq
                ene = eij * t3 * (t3 - 1.0)
                return a + jnp.where(r2 <= c2, ene, 0.0)

            return lax.fori_loop(
                0, steps // 8, block, jnp.zeros((LANES,), jnp.float32)
            )

        zero16 = jnp.zeros((LANES,), jnp.float32)

        fire_idx(jnp.int32(0), 0)
        fire_gath(jnp.int32(0), 0)
        fire_idx(jnp.int32(1), 1)

        def body(i, acc):
            t0 = 2 * i
            t1 = t0 + 1
            fire_gath(t1, 1)
            wait_gath(t0, 0)
            fire_idx(t0 + 2, 0)
            acc = acc + jnp.where(t0 < n_mine, compute(0), zero16)
            fire_gath(t0 + 2, 0)
            wait_gath(t1, 1)
            fire_idx(t1 + 2, 1)
            acc = acc + jnp.where(t1 < n_mine, compute(1), zero16)
            return acc

        acc = lax.fori_loop(0, n_it, body, zero16)

        accv[...] = acc
        pltpu.sync_copy(accv, shared.at[pl.ds(sid * LANES, LANES)])
        plsc.subcore_barrier()

        @pl.when(sid == 0)
        def _():
            pltpu.sync_copy(shared, redv)
            tot = redv[pl.ds(0, LANES)]
            for k in range(1, NS):
                tot = tot + redv[pl.ds(k * LANES, LANES)]
            s_val = jnp.sum(tot)
            accv[...] = jnp.where(iota == 0, s_val, 0.0)
            pltpu.sync_copy(accv.at[pl.ds(0, 8)], out.at[pl.ds(cid * 8, 8)])

    return lj


def kernel(coords, pairs, box, sigma, epsilon, cutoff):
    n_nodes = coords.shape[0]
    n_pairs = pairs.shape[0]
    half_sigma = (sigma * 0.5)[:, None]
    two_sqeps = (2.0 * jnp.sqrt(epsilon))[:, None]
    pad = jnp.zeros((n_nodes, ROW_W - 5), jnp.float32)
    table = jnp.concatenate(
        [coords.astype(jnp.float32), half_sigma, two_sqeps, pad], axis=1
    )
    box = box.astype(jnp.float32)
    box_inv = jnp.linalg.inv(box)
    c2 = (jnp.asarray(cutoff, jnp.float32) ** 2).reshape(1)
    colsum = jnp.max(jnp.sum(jnp.abs(box_inv), axis=0))
    thr = (0.5 * (1.0 - 1e-5) / colsum).reshape(1).astype(jnp.float32)
    params = jnp.concatenate(
        [box_inv.reshape(9), box.reshape(9), c2, thr,
         jnp.zeros(12, jnp.float32)]
    ).astype(jnp.float32)
    n_rows = n_pairs * 2 // 128
    prows = (
        pairs.astype(jnp.int32)
        .T.reshape(2, n_pairs // 128, 128)
        .transpose(1, 0, 2)
        .reshape(n_rows * 128)
    )
    out = _make_kernel(n_rows)(table, prows, params)
    return out[0] + out[8]


# speedup vs baseline: 1.2235x; 1.2235x over previous
import functools

import jax
import jax.numpy as jnp
from jax import lax
from jax.experimental import pallas as pl
from jax.experimental.pallas import tpu as pltpu
from jax.experimental.pallas import tpu_sc as plsc

NC = 2
NS = 16
LANES = 16
ROW_W = 8
CHUNK_ROWS = 16


@functools.lru_cache(maxsize=None)
def _make_kernel(n_rows: int):
    n_workers = NC * NS
    assert n_rows % CHUNK_ROWS == 0
    n_chunks = n_rows // CHUNK_ROWS
    ent = CHUNK_ROWS * 128
    steps = ent // (2 * LANES)
    max_mine = n_chunks // n_workers + (1 if n_chunks % n_workers else 0)
    n_it = (max_mine + 1) // 2

    mesh = plsc.VectorSubcoreMesh(core_axis_name="c", subcore_axis_name="s")

    @functools.partial(
        pl.kernel,
        out_type=jax.ShapeDtypeStruct((2 * 8,), jnp.float32),
        mesh=mesh,
        compiler_params=pltpu.CompilerParams(
            needs_layout_passes=False, use_tc_tiling_on_sc=False
        ),
        scratch_types=[
            pltpu.VMEM((2 * ent,), jnp.int32),
            pltpu.VMEM((2 * ent, ROW_W), jnp.float32),
            pltpu.VMEM((32,), jnp.float32),
            pltpu.VMEM((LANES,), jnp.float32),
            pltpu.VMEM_SHARED((NS * LANES,), jnp.float32),
            pltpu.VMEM((NS * LANES,), jnp.float32),
            pltpu.SemaphoreType.DMA,
            pltpu.SemaphoreType.DMA,
            pltpu.SemaphoreType.DMA,
            pltpu.SemaphoreType.DMA,
        ],
    )
    def lj(table, prows, params, out, idxv, rowsv, pv, accv, shared, redv,
           isem0, isem1, gsem0, gsem1):
        cid = lax.axis_index("c")
        sid = lax.axis_index("s")
        wid = sid * NC + cid
        isem = (isem0, isem1)
        gsem = (gsem0, gsem1)

        pltpu.sync_copy(params, pv)
        pva = pv[pl.ds(0, LANES)]
        pvb = pv[pl.ds(LANES, LANES)]
        bi = [pva[i] for i in range(9)]
        gg = [pva[9 + i] for i in range(6)]
        c2 = pva[15]

        iota = lax.iota(jnp.int32, LANES)

        n_mine = n_chunks // n_workers + jnp.where(
            wid < n_chunks % n_workers, 1, 0
        )

        def fire_idx(t, slot):
            @pl.when(t < n_mine)
            def _():
                g = wid + t * n_workers
                pltpu.async_copy(
                    prows.at[pl.ds(g * ent, ent)],
                    idxv.at[pl.ds(slot * ent, ent)],
                    isem[slot],
                )

        def fire_gath(t, slot):
            @pl.when(t < n_mine)
            def _():
                pltpu.make_async_copy(
                    prows.at[pl.ds(0, ent)],
                    idxv.at[pl.ds(slot * ent, ent)],
                    isem[slot],
                ).wait()
                for k in range(CHUNK_ROWS):
                    pltpu.async_copy(
                        table.at[idxv.at[pl.ds(slot * ent + k * 128, 128)]],
                        rowsv.at[pl.ds(slot * ent + k * 128, 128)],
                        gsem[slot],
                    )

        def wait_gath(t, slot):
            @pl.when(t < n_mine)
            def _():
                pltpu.make_async_copy(
                    table.at[pl.ds(0, ent)],
                    rowsv.at[pl.ds(slot * ent, ent)],
                    gsem[slot],
                ).wait()

        def compute(slot):
            base = slot * ent

            def block(m, accs):
                base_b = base + m * 256
                a0, a1 = accs
                for u in range(0, 8, 2):
                    a0 = step(base_b + u * LANES, a0)
                    a1 = step(base_b + (u + 1) * LANES, a1)
                return (a0, a1)

            def step(b0, a):
                ri = iota + b0
                rj = ri + 128
                c0 = jnp.zeros((LANES,), jnp.int32)
                xi = plsc.load_gather(rowsv, [ri, c0])
                yi = plsc.load_gather(rowsv, [ri, c0 + 1])
                zi = plsc.load_gather(rowsv, [ri, c0 + 2])
                si = plsc.load_gather(rowsv, [ri, c0 + 3])
                ei = plsc.load_gather(rowsv, [ri, c0 + 4])
                xj = plsc.load_gather(rowsv, [rj, c0])
                yj = plsc.load_gather(rowsv, [rj, c0 + 1])
                zj = plsc.load_gather(rowsv, [rj, c0 + 2])
                sj = plsc.load_gather(rowsv, [rj, c0 + 3])
                ej = plsc.load_gather(rowsv, [rj, c0 + 4])

                dx = xi - xj
                dy = yi - yj
                dz = zi - zj
                d0 = dx * bi[0] + dy * bi[3] + dz * bi[6]
                d1 = dx * bi[1] + dy * bi[4] + dz * bi[7]
                d2 = dx * bi[2] + dy * bi[5] + dz * bi[8]

                def wrap(v):
                    y = v + 0.5
                    tf = y.astype(jnp.int32).astype(jnp.float32)
                    fl = tf - jnp.where(y < tf, 1.0, 0.0)
                    return v - fl

                w0 = wrap(d0)
                w1 = wrap(d1)
                w2 = wrap(d2)
                r2 = (
                    gg[0] * (w0 * w0)
                    + gg[1] * (w1 * w1)
                    + gg[2] * (w2 * w2)
                    + gg[3] * (w0 * w1)
                    + gg[4] * (w0 * w2)
                    + gg[5] * (w1 * w2)
                )

                sij = si + sj
                eij = ei * ej
                tq = (sij * sij) / r2
                t3 = tq * tq * tq
                ene = eij * t3 * (t3 - 1.0)
                return a + jnp.where(r2 <= c2, ene, 0.0)

            z = jnp.zeros((LANES,), jnp.float32)
            a0, a1 = lax.fori_loop(0, steps // 8, block, (z, z))
            return a0 + a1

        zero16 = jnp.zeros((LANES,), jnp.float32)

        fire_idx(jnp.int32(0), 0)
        fire_gath(jnp.int32(0), 0)
        fire_idx(jnp.int32(1), 1)

        def body(i, acc):
            t0 = 2 * i
            t1 = t0 + 1
            fire_gath(t1, 1)
            wait_gath(t0, 0)
            fire_idx(t0 + 2, 0)
            acc = acc + jnp.where(t0 < n_mine, compute(0), zero16)
            fire_gath(t0 + 2, 0)
            wait_gath(t1, 1)
            fire_idx(t1 + 2, 1)
            acc = acc + jnp.where(t1 < n_mine, compute(1), zero16)
            return acc

        acc = lax.fori_loop(0, n_it, body, zero16)

        accv[...] = acc
        pltpu.sync_copy(accv, shared.at[pl.ds(sid * LANES, LANES)])
        plsc.subcore_barrier()

        @pl.when(sid == 0)
        def _():
            pltpu.sync_copy(shared, redv)
            tot = redv[pl.ds(0, LANES)]
            for k in range(1, NS):
                tot = tot + redv[pl.ds(k * LANES, LANES)]
            s_val = jnp.sum(tot)
            accv[...] = jnp.where(iota == 0, s_val, 0.0)
            pltpu.sync_copy(accv.at[pl.ds(0, 8)], out.at[pl.ds(cid * 8, 8)])

    return lj


def kernel(coords, pairs, box, sigma, epsilon, cutoff):
    n_nodes = coords.shape[0]
    n_pairs = pairs.shape[0]
    half_sigma = (sigma * 0.5)[:, None]
    two_sqeps = (2.0 * jnp.sqrt(epsilon))[:, None]
    pad = jnp.zeros((n_nodes, ROW_W - 5), jnp.float32)
    table = jnp.concatenate(
        [coords.astype(jnp.float32), half_sigma, two_sqeps, pad], axis=1
    )
    box = box.astype(jnp.float32)
    box_inv = jnp.linalg.inv(box)
    c2 = (jnp.asarray(cutoff, jnp.float32) ** 2).reshape(1)
    g = box @ box.T
    gram = jnp.stack(
        [g[0, 0], g[1, 1], g[2, 2], 2 * g[0, 1], 2 * g[0, 2], 2 * g[1, 2]]
    )
    params = jnp.concatenate(
        [box_inv.reshape(9), gram, c2, jnp.zeros(16, jnp.float32)]
    ).astype(jnp.float32)
    n_rows = n_pairs * 2 // 128
    prows = (
        pairs.astype(jnp.int32)
        .T.reshape(2, n_pairs // 128, 128)
        .transpose(1, 0, 2)
        .reshape(n_rows * 128)
    )
    out = _make_kernel(n_rows)(table, prows, params)
    return out[0] + out[8]
